# baseline (speedup 1.0000x reference)
import functools
import math

import jax
import jax.numpy as jnp
from jax.experimental import pallas as pl
from jax.experimental.pallas import tpu as pltpu

HID = 128
NFB = 64
NL = 3
NH = 8
HD = 16
TOKEN_SIZE = 1024
N_NODES = 10000
N_POLY = 500


def _linear(p, x):
    y = x @ p["w"]
    if "b" in p:
        y = y + p["b"]
    return y


def _layernorm(p, x, eps=1e-5):
    m = jnp.mean(x, axis=-1, keepdims=True)
    v = jnp.mean((x - m) ** 2, axis=-1, keepdims=True)
    return (x - m) / jnp.sqrt(v + eps) * p["g"] + p["b"]


def _wrap_angle(a):
    return (a + math.pi) % (2 * math.pi) - math.pi


def _mlp3(p, x):
    h = _linear(p["l1"], x)
    h = _layernorm(p["ln"], h)
    h = jax.nn.relu(h)
    return _linear(p["l2"], h)


def _fourier_embedding(p, x):
    xe = x[..., None] * p["freqs"] * 2.0 * math.pi
    xcat = jnp.concatenate([jnp.cos(xe), jnp.sin(xe), x[..., None]], axis=-1)
    embs = [_mlp3(p["mlps"][i], xcat[:, i]) for i in range(x.shape[-1])]
    out = jnp.stack(embs).sum(axis=0)
    out = _layernorm(p["to_out"]["ln"], out)
    out = jax.nn.relu(out)
    return _linear(p["to_out"]["lin"], out)


def _segment_softmax(sim, index, num_segments):
    smax = jax.ops.segment_max(sim, index, num_segments=num_segments)
    smax = jnp.where(jnp.isfinite(smax), smax, 0.0)
    ex = jnp.exp(sim - smax[index])
    denom = jax.ops.segment_sum(ex, index, num_segments=num_segments)
    return ex / (denom[index] + 1e-16)


def _attention_layer(p, x, r, edge_index, num_nodes):
    src, dst = edge_index[0], edge_index[1]
    xn = _layernorm(p["pre_x"], x)
    rn = _layernorm(p["pre_r"], r)
    q = _linear(p["to_q"], xn).reshape(-1, NH, HD)
    k = _linear(p["to_k"], xn).reshape(-1, NH, HD)
    v = _linear(p["to_v"], xn).reshape(-1, NH, HD)
    k_j = k[src] + _linear(p["to_k_r"], rn).reshape(-1, NH, HD)
    v_j = v[src] + _linear(p["to_v_r"], rn).reshape(-1, NH, HD)
    q_i = q[dst]
    sim = (q_i * k_j).sum(axis=-1) * (HD ** -0.5)
    attn = _segment_softmax(sim, dst, num_nodes)
    agg = jax.ops.segment_sum(v_j * attn[..., None], dst, num_segments=num_nodes)
    inputs = agg.reshape(-1, NH * HD)
    g = jax.nn.sigmoid(_linear(p["to_g"], jnp.concatenate([inputs, xn], axis=-1)))
    agg = inputs + g * (_linear(p["to_s"], xn) - inputs)
    x = x + _linear(p["to_out"], agg)
    h = _layernorm(p["ff_pre"], x)
    h = jax.nn.relu(_linear(p["ff1"], h))
    h = _linear(p["ff2"], h)
    return x + h


def _head_body(x_ref, w1_ref, b1_ref, g_ref, bln_ref, w2_ref, b2_ref, o_ref):
    x = x_ref[...]
    h = jnp.dot(x, w1_ref[...], preferred_element_type=jnp.float32) + b1_ref[...]
    m = jnp.mean(h, axis=-1, keepdims=True)
    v = jnp.mean((h - m) ** 2, axis=-1, keepdims=True)
    h = (h - m) / jnp.sqrt(v + 1e-5) * g_ref[...] + bln_ref[...]
    h = jax.nn.relu(h)
    o_ref[...] = jnp.dot(h, w2_ref[...], preferred_element_type=jnp.float32) + b2_ref[...]


def _head_pallas(p, x):
    n = x.shape[0]
    blk = 1000
    grid = n // blk
    return pl.pallas_call(
        _head_body,
        grid=(grid,),
        in_specs=[
            pl.BlockSpec((blk, HID), lambda i: (i, 0)),
            pl.BlockSpec((HID, HID), lambda i: (0, 0)),
            pl.BlockSpec((HID,), lambda i: (0,)),
            pl.BlockSpec((HID,), lambda i: (0,)),
            pl.BlockSpec((HID,), lambda i: (0,)),
            pl.BlockSpec((HID, TOKEN_SIZE), lambda i: (0, 0)),
            pl.BlockSpec((TOKEN_SIZE,), lambda i: (0,)),
        ],
        out_specs=pl.BlockSpec((blk, TOKEN_SIZE), lambda i: (i, 0)),
        out_shape=jax.ShapeDtypeStruct((n, TOKEN_SIZE), jnp.float32),
    )(x, p["l1"]["w"], p["l1"]["b"], p["ln"]["g"], p["ln"]["b"], p["l2"]["w"], p["l2"]["b"])


def kernel(position, orientation, traj_src, params, pt_valid_mask, pt_pred_mask, pt_target_mask, token_idx, pt_type, pl_type, light_type, token2pl, edge_index_pt2pt):
    pos_pt = position[:, :2]
    orient_pt = orientation
    orient_vector_pt = jnp.stack([jnp.cos(orient_pt), jnp.sin(orient_pt)], axis=-1)
    flat = traj_src.astype(jnp.float32).reshape(traj_src.shape[0], -1)
    pt_token_emb_src = _mlp3(params["token_emb"], flat)
    x_pt = pt_token_emb_src[token_idx]
    token_light_type = light_type[token2pl[1]]
    cat_embs = (params["type_pt_emb"][pt_type] + params["polygon_type_emb"][pl_type]
                + params["light_pl_emb"][token_light_type])
    x_pt = x_pt + cat_embs
    src, dst = edge_index_pt2pt[0], edge_index_pt2pt[1]
    rel_pos = pos_pt[src] - pos_pt[dst]
    rel_orient = _wrap_angle(orient_pt[src] - orient_pt[dst])
    ov_dst = orient_vector_pt[dst]
    ang = jnp.arctan2(ov_dst[:, 0] * rel_pos[:, 1] - ov_dst[:, 1] * rel_pos[:, 0],
                      (ov_dst * rel_pos).sum(axis=-1))
    r = jnp.stack([jnp.linalg.norm(rel_pos, axis=-1), ang, rel_orient], axis=-1)
    r = _fourier_embedding(params["r_pt2pt_emb"], r)
    for i in range(NL):
        x_pt = _attention_layer(params["layers"][i], x_pt, r, edge_index_pt2pt, x_pt.shape[0])
    next_token_prob = _head_pallas(params["head"], jnp.where(pt_pred_mask[:, None], x_pt, 0.0))
    _, next_token_idx = jax.lax.top_k(next_token_prob, 10)
    next_token_index_gt = jnp.where(pt_target_mask, token_idx, 0)
    return x_pt, next_token_idx, next_token_prob, next_token_index_gt, pt_pred_mask


# speedup vs baseline: 1.0322x; 1.0322x over previous
import math

import jax
import jax.numpy as jnp
from jax import lax
from jax.experimental import pallas as pl

HID = 128
NFB = 64
NL = 3
NH = 8
HD = 16
TOKEN_SIZE = 1024
N_POLY = 500

f32 = jnp.float32


def _linear(p, x):
    y = x @ p["w"]
    if "b" in p:
        y = y + p["b"]
    return y


def _layernorm(p, x, eps=1e-5):
    m = jnp.mean(x, axis=-1, keepdims=True)
    v = jnp.mean((x - m) ** 2, axis=-1, keepdims=True)
    return (x - m) / jnp.sqrt(v + eps) * p["g"] + p["b"]


def _wrap_angle(a):
    return (a + math.pi) % (2 * math.pi) - math.pi


def _mlp3(p, x):
    h = _linear(p["l1"], x)
    h = _layernorm(p["ln"], h)
    h = jax.nn.relu(h)
    return _linear(p["l2"], h)


def _fourier_embedding(p, x):
    xe = x[..., None] * p["freqs"] * 2.0 * math.pi
    xcat = jnp.concatenate([jnp.cos(xe), jnp.sin(xe), x[..., None]], axis=-1)
    embs = [_mlp3(p["mlps"][i], xcat[:, i]) for i in range(x.shape[-1])]
    out = jnp.stack(embs).sum(axis=0)
    out = _layernorm(p["to_out"]["ln"], out)
    out = jax.nn.relu(out)
    return _linear(p["to_out"]["lin"], out)


def _segment_softmax(sim, index, num_segments):
    smax = jax.ops.segment_max(sim, index, num_segments=num_segments)
    smax = jnp.where(jnp.isfinite(smax), smax, 0.0)
    ex = jnp.exp(sim - smax[index])
    denom = jax.ops.segment_sum(ex, index, num_segments=num_segments)
    return ex / (denom[index] + 1e-16)


def _attention_layer(p, x, r, edge_index, num_nodes):
    src, dst = edge_index[0], edge_index[1]
    xn = _layernorm(p["pre_x"], x)
    rn = _layernorm(p["pre_r"], r)
    q = _linear(p["to_q"], xn).reshape(-1, NH, HD)
    k = _linear(p["to_k"], xn).reshape(-1, NH, HD)
    v = _linear(p["to_v"], xn).reshape(-1, NH, HD)
    k_j = k[src] + _linear(p["to_k_r"], rn).reshape(-1, NH, HD)
    v_j = v[src] + _linear(p["to_v_r"], rn).reshape(-1, NH, HD)
    q_i = q[dst]
    sim = (q_i * k_j).sum(axis=-1) * (HD ** -0.5)
    attn = _segment_softmax(sim, dst, num_nodes)
    agg = jax.ops.segment_sum(v_j * attn[..., None], dst, num_segments=num_nodes)
    inputs = agg.reshape(-1, NH * HD)
    g = jax.nn.sigmoid(_linear(p["to_g"], jnp.concatenate([inputs, xn], axis=-1)))
    agg = inputs + g * (_linear(p["to_s"], xn) - inputs)
    x = x + _linear(p["to_out"], agg)
    h = _layernorm(p["ff_pre"], x)
    h = jax.nn.relu(_linear(p["ff1"], h))
    h = _linear(p["ff2"], h)
    return x + h


BLK = 1000


def _head_body(x_ref, w1_ref, b1_ref, g_ref, bln_ref, w2_ref, b2_ref,
               p_ref, i_ref):
    h = jnp.dot(x_ref[...], w1_ref[...], preferred_element_type=f32) + b1_ref[...]
    m = jnp.mean(h, axis=-1, keepdims=True)
    v = jnp.mean((h - m) ** 2, axis=-1, keepdims=True)
    h = (h - m) / jnp.sqrt(v + 1e-5) * g_ref[...] + bln_ref[...]
    logits = (jnp.dot(jax.nn.relu(h), w2_ref[...], preferred_element_type=f32)
              + b2_ref[...])
    p_ref[...] = logits
    iota = lax.broadcasted_iota(jnp.int32, (BLK, TOKEN_SIZE), 1)
    cols = []
    l = logits
    for _ in range(10):
        mx = jnp.max(l, axis=1, keepdims=True)
        idx = jnp.min(jnp.where(l == mx, iota, jnp.int32(2 ** 30)),
                      axis=1, keepdims=True)
        cols.append(idx)
        l = jnp.where(iota == idx, -jnp.inf, l)
    i_ref[...] = jnp.concatenate(cols + [jnp.zeros((BLK, 118), jnp.int32)],
                                 axis=1)


def _head_pallas(p, x):
    n = x.shape[0]
    wsp = lambda s: pl.BlockSpec(s, lambda i, _n=len(s): (0,) * _n)
    return pl.pallas_call(
        _head_body, grid=(n // BLK,),
        in_specs=[pl.BlockSpec((BLK, HID), lambda i: (i, 0)),
                  wsp((HID, HID)), wsp((1, HID)), wsp((1, HID)),
                  wsp((1, HID)), wsp((HID, TOKEN_SIZE)), wsp((1, TOKEN_SIZE))],
        out_specs=[pl.BlockSpec((BLK, TOKEN_SIZE), lambda i: (i, 0)),
                   pl.BlockSpec((BLK, 128), lambda i: (i, 0))],
        out_shape=[jax.ShapeDtypeStruct((n, TOKEN_SIZE), f32),
                   jax.ShapeDtypeStruct((n, 128), jnp.int32)],
    )(x, p["l1"]["w"], p["l1"]["b"][None], p["ln"]["g"][None],
      p["ln"]["b"][None], p["l2"]["w"], p["l2"]["b"][None])


def kernel(position, orientation, traj_src, params, pt_valid_mask,
           pt_pred_mask, pt_target_mask, token_idx, pt_type, pl_type,
           light_type, token2pl, edge_index_pt2pt):
    pos_pt = position[:, :2]
    orient_pt = orientation
    orient_vector_pt = jnp.stack([jnp.cos(orient_pt), jnp.sin(orient_pt)], axis=-1)
    flat = traj_src.astype(f32).reshape(traj_src.shape[0], -1)
    pt_token_emb_src = _mlp3(params["token_emb"], flat)
    x_pt = pt_token_emb_src[token_idx]
    token_light_type = light_type[token2pl[1]]
    cat_embs = jnp.stack([params["type_pt_emb"][pt_type],
                          params["polygon_type_emb"][pl_type],
                          params["light_pl_emb"][token_light_type]]).sum(axis=0)
    x_pt = x_pt + cat_embs
    src, dst = edge_index_pt2pt[0], edge_index_pt2pt[1]
    rel_pos = pos_pt[src] - pos_pt[dst]
    rel_orient = _wrap_angle(orient_pt[src] - orient_pt[dst])
    ov_dst = orient_vector_pt[dst]
    ang = jnp.arctan2(
        ov_dst[..., 0] * rel_pos[..., 1] - ov_dst[..., 1] * rel_pos[..., 0],
        (ov_dst * rel_pos[:, :2]).sum(axis=-1))
    r = jnp.stack([jnp.linalg.norm(rel_pos[:, :2], axis=-1), ang, rel_orient],
                  axis=-1)
    r = _fourier_embedding(params["r_pt2pt_emb"], r)
    for i in range(NL):
        x_pt = _attention_layer(params["layers"][i], x_pt, r,
                                edge_index_pt2pt, x_pt.shape[0])
    probs, idx128 = _head_pallas(params["head"],
                                 jnp.where(pt_pred_mask[:, None], x_pt, 0.0))
    next_token_idx = idx128[:, :10]
    next_token_index_gt = jnp.where(pt_target_mask, token_idx, 0)
    return x_pt, next_token_idx, probs, next_token_index_gt, pt_pred_mask
